# Initial kernel scaffold; baseline (speedup 1.0000x reference)
import jax
import jax.numpy as jnp
from jax.experimental import pallas as pl


def kernel(inputs, edge_index, W_in, b_in, W_mid, b_mid, W_out, b_out):
    raise NotImplementedError("write your pallas kernel here")


# speedup vs baseline: 1.4002x; 1.4002x over previous
import functools

import jax
import jax.numpy as jnp
from jax import lax
from jax.experimental import pallas as pl
from jax.experimental.pallas import tpu as pltpu
from jax.experimental.pallas import tpu_sc as plsc

N = 10000
E = 160000
D = 256
SW = 4
NSL = D // SW
NC = 2
NS = 16
NW = NC * NS
SUB = NSL // NW
EPAD = 163840
SB = 2048
NB = EPAD // SB
NP = 10112
DUMP = 10000
CNP = 10240
CSTR = CNP // NW
BM = 1000
L = 16


def _seg_body(tab_h, src_h, dst_h, sums, cnt_out,
              table, acc, cnt, src_st, dst_st, isem):
    c = lax.axis_index("c")
    s = lax.axis_index("s")
    wid = s * NC + c

    iota = lax.iota(jnp.int32, L)
    zv = jnp.zeros((L,), jnp.float32)
    ov = jnp.ones((L,), jnp.float32)
    row4 = iota // SW
    col4 = iota % SW
    qvs = [jnp.full((L,), q, jnp.int32) for q in range(SW)]

    def zc(i, carry):
        cnt[pl.ds(i * L, L)] = zv
        return carry
    lax.fori_loop(0, CNP // L, zc, 0)

    for h in range(SUB):
        sl = h * NW + wid
        pltpu.sync_copy(tab_h.at[sl], table)

        def za(i, carry):
            acc[pl.ds(i * L, L)] = zv
            return carry
        lax.fori_loop(0, NP * SW // L, za, 0)

        pltpu.async_copy(src_h.at[pl.ds(0, SB)], src_st.at[0], isem)
        pltpu.async_copy(dst_h.at[pl.ds(0, SB)], dst_st.at[0], isem)

        def blk_step(b, carry):
            p = lax.rem(b, 2)
            pltpu.make_async_copy(src_h.at[pl.ds(0, SB)], src_st.at[p], isem).wait()
            pltpu.make_async_copy(dst_h.at[pl.ds(0, SB)], dst_st.at[p], isem).wait()

            @pl.when(b + 1 < NB)
            def _():
                off = pl.multiple_of((b + 1) * SB, 8)
                pltpu.async_copy(src_h.at[pl.ds(off, SB)], src_st.at[1 - p], isem)
                pltpu.async_copy(dst_h.at[pl.ds(off, SB)], dst_st.at[1 - p], isem)

            def group_step(g, carry2):
                srcv = src_st[p, pl.ds(g * L, L)]
                dstv = dst_st[p, pl.ds(g * L, L)]
                srcb = srcv * SW
                dstb = dstv * SW
                for q in range(SW):
                    vals = plsc.load_gather(table, [srcb + qvs[q]])
                    plsc.addupdate_scatter(acc, [dstb + qvs[q]], vals)
                if h == 0:
                    plsc.addupdate_scatter(cnt, [dstv], ov)
                return carry2

            lax.fori_loop(0, SB // L, group_step, 0)
            return carry

        lax.fori_loop(0, NB, blk_step, 0)

        pltpu.sync_copy(acc, sums.at[sl])

    cs = pl.ds(wid * CSTR, CSTR)
    pltpu.sync_copy(cnt.at[cs], cnt_out.at[cs])


def _make_seg_sum():
    f32 = jnp.float32
    out_type = [jax.ShapeDtypeStruct((NSL, NP * SW), f32),
                jax.ShapeDtypeStruct((CNP,), f32)]
    scratch = [
        pltpu.VMEM((N * SW,), f32),
        pltpu.VMEM((NP * SW,), f32),
        pltpu.VMEM((CNP,), f32),
        pltpu.VMEM((2, SB), jnp.int32),
        pltpu.VMEM((2, SB), jnp.int32),
        pltpu.SemaphoreType.DMA,
    ]
    mesh = plsc.VectorSubcoreMesh(core_axis_name="c", subcore_axis_name="s",
                                  num_cores=NC, num_subcores=NS)
    return pl.kernel(_seg_body, out_type=out_type, mesh=mesh,
                     scratch_types=scratch,
                     compiler_params=pltpu.CompilerParams(needs_layout_passes=False))


def _lin_body(x_ref, cnt_ref, w_ref, b_ref, o_ref):
    inv = 1.0 / jnp.maximum(cnt_ref[...][:, 0:1], 1.0)
    x = x_ref[...] * inv
    y = jnp.dot(x, w_ref[...], preferred_element_type=jnp.float32)
    o_ref[...] = jnp.maximum(y + b_ref[...], 0.0)


def _lin_layer(x, cnt, w, b):
    return pl.pallas_call(
        _lin_body,
        grid=(N // BM,),
        in_specs=[pl.BlockSpec((BM, D), lambda i: (i, 0)),
                  pl.BlockSpec((BM, 8), lambda i: (i, 0)),
                  pl.BlockSpec((D, D), lambda i: (0, 0)),
                  pl.BlockSpec((1, D), lambda i: (0, 0))],
        out_specs=pl.BlockSpec((BM, D), lambda i: (i, 0)),
        out_shape=jax.ShapeDtypeStruct((N, D), jnp.float32),
    )(x, cnt, w, b.reshape(1, D))


def _pad_flat(idx, fill):
    pad = jnp.full((EPAD - E,), fill, dtype=idx.dtype)
    return jnp.concatenate([idx, pad])


def _to_slivers(m):
    return m.reshape(N, NSL, SW).transpose(1, 0, 2).reshape(NSL, N * SW)


def _from_slivers(s):
    return s.reshape(NSL, NP, SW).transpose(1, 0, 2).reshape(NP, D)[:N]


def kernel(inputs, edge_index, W_in, b_in, W_mid, b_mid, W_out, b_out):
    news = _pad_flat(edge_index[0], DUMP)
    news_s = _pad_flat(edge_index[0], 0)
    stock = _pad_flat(edge_index[1], DUMP)
    stock_s = _pad_flat(edge_index[1], 0)

    ones8 = jnp.ones((N, 8), jnp.float32)
    seg = _make_seg_sum()

    m = _lin_layer(inputs, ones8, W_in, b_in)
    s1, cnt1 = seg(_to_slivers(m), stock_s, news)
    cnt1b = jnp.broadcast_to(cnt1[:N, None], (N, 8))
    c = _lin_layer(_from_slivers(s1), cnt1b, W_mid, b_mid)
    s2, cnt2 = seg(_to_slivers(c), news_s, stock)
    cnt2b = jnp.broadcast_to(cnt2[:N, None], (N, 8))
    out = _lin_layer(_from_slivers(s2), cnt2b, W_out, b_out)
    return out
